# Initial kernel scaffold; baseline (speedup 1.0000x reference)
import jax
import jax.numpy as jnp
from jax.experimental import pallas as pl


def kernel(z, node_feat, edge_index, edge_weight, edge_attr, emb_table, dp_w, dp_b, comb_w, comb_b):
    raise NotImplementedError("write your pallas kernel here")


# speedup vs baseline: 1.6750x; 1.6750x over previous
import functools

import jax
import jax.numpy as jnp
from jax import lax
from jax.experimental import pallas as pl
from jax.experimental.pallas import tpu as pltpu
from jax.experimental.pallas import tpu_sc as plsc

N_NODES = 10000
N_EDGES = 320000
HIDDEN = 128
NUM_RBF = 64
CUTOFF = 5.0

NC = 2
NS = 16
NW = NC * NS
EPW = N_EDGES // NW
BL = 80
JB = EPW // BL
NPAD = 10240
ROWS_PER_TILE = NPAD // NS

@functools.lru_cache(maxsize=None)
def _build_zdst():
    mesh = plsc.VectorSubcoreMesh(core_axis_name="c", subcore_axis_name="s",
                                  num_cores=NC, num_subcores=NS)

    @functools.partial(
        pl.kernel,
        out_type=jax.ShapeDtypeStruct((NW, EPW), jnp.int32),
        mesh=mesh,
        scratch_types=[
            pltpu.VMEM((N_NODES,), jnp.int32),
            pltpu.VMEM((EPW,), jnp.int32),
            pltpu.VMEM((EPW,), jnp.int32),
        ],
        compiler_params=pltpu.CompilerParams(needs_layout_passes=False),
    )
    def zdst_kernel(z_hbm, dst_hbm, out_hbm, z_v, dst_v, out_v):
        wid = lax.axis_index("s") * NC + lax.axis_index("c")
        pltpu.sync_copy(z_hbm, z_v)
        pltpu.sync_copy(dst_hbm.at[wid], dst_v)

        def body(i, carry):
            idx = dst_v[pl.ds(i * 16, 16)]
            out_v[pl.ds(i * 16, 16)] = plsc.load_gather(z_v, [idx])
            return carry

        lax.fori_loop(0, EPW // 16, body, 0)
        pltpu.sync_copy(out_v, out_hbm.at[wid])

    return zdst_kernel


def _zdst_sc(z, dst):
    return _build_zdst()(z, dst)


_MSG_B = 2000


def _msg_body(ea_ref, ew_ref, zd_ref, dpw_ref, dpb_ref, emb_ref, out_ref):
    w = jnp.dot(ea_ref[...], dpw_ref[...], preferred_element_type=jnp.float32)
    ew = ew_ref[...]
    c = jnp.where(ew < CUTOFF, 0.5 * (jnp.cos(ew * (jnp.pi / CUTOFF)) + 1.0), 0.0)
    zd = zd_ref[...]
    oh = (zd == lax.broadcasted_iota(jnp.int32, (_MSG_B, 128), 1)).astype(jnp.float32)
    g = jnp.dot(oh, emb_ref[...], preferred_element_type=jnp.float32)
    out_ref[...] = (w + dpb_ref[...]) * c * g


def _msg_tc(ea, ew2, zd2, dpw_t, dpb2, emb_pad):
    nblk = N_EDGES // _MSG_B
    return pl.pallas_call(
        _msg_body,
        grid=(nblk,),
        in_specs=[
            pl.BlockSpec((_MSG_B, NUM_RBF), lambda i: (i, 0)),
            pl.BlockSpec((_MSG_B, 1), lambda i: (i, 0)),
            pl.BlockSpec((_MSG_B, 1), lambda i: (i, 0)),
            pl.BlockSpec((NUM_RBF, HIDDEN), lambda i: (0, 0)),
            pl.BlockSpec((1, HIDDEN), lambda i: (0, 0)),
            pl.BlockSpec((128, HIDDEN), lambda i: (0, 0)),
        ],
        out_specs=pl.BlockSpec((_MSG_B, HIDDEN), lambda i: (i, 0)),
        out_shape=jax.ShapeDtypeStruct((N_EDGES, HIDDEN), jnp.float32),
        compiler_params=pltpu.CompilerParams(
            dimension_semantics=("arbitrary",),
        ),
    )(ea, ew2, zd2, dpw_t, dpb2, emb_pad)


@functools.lru_cache(maxsize=None)
def _build_scatter():
    mesh = plsc.VectorSubcoreMesh(core_axis_name="c", subcore_axis_name="s",
                                  num_cores=NC, num_subcores=NS)

    @functools.partial(
        pl.kernel,
        out_type=jax.ShapeDtypeStruct((NC * NPAD, HIDDEN), jnp.float32),
        mesh=mesh,
        scratch_types=[
            pltpu.VMEM_SHARED((NPAD, HIDDEN), jnp.float32),
            pltpu.VMEM((JB, BL), jnp.int32),
            pltpu.VMEM((BL, HIDDEN), jnp.float32),
        ],
        compiler_params=pltpu.CompilerParams(needs_layout_passes=False),
    )
    def scatter_kernel(msg_hbm, src_hbm, zeros_hbm, out_hbm, agg_sh, idx_v, buf):
        c = lax.axis_index("c")
        s = lax.axis_index("s")
        wid = s * NC + c
        rb = s * ROWS_PER_TILE
        pltpu.sync_copy(zeros_hbm.at[pl.ds(rb, ROWS_PER_TILE)],
                        agg_sh.at[pl.ds(rb, ROWS_PER_TILE)])
        pltpu.sync_copy(src_hbm.at[wid], idx_v)
        plsc.subcore_barrier()
        ebase = wid * EPW

        def body(j, carry):
            pltpu.sync_copy(msg_hbm.at[pl.ds(ebase + j * BL, BL)], buf)
            pltpu.sync_copy(buf, agg_sh.at[idx_v.at[j]], add=True)
            return carry

        lax.fori_loop(0, JB, body, 0)
        plsc.subcore_barrier()
        pltpu.sync_copy(agg_sh.at[pl.ds(rb, ROWS_PER_TILE)],
                        out_hbm.at[pl.ds(c * NPAD + rb, ROWS_PER_TILE)])

    return scatter_kernel


def _scatter_sc(msg, src3, zeros):
    return _build_scatter()(msg, src3, zeros)


_OUT_B = 1000


def _out_body(nf_ref, a0_ref, a1_ref, cw1_ref, cw2_ref, cb_ref, out_ref):
    agg = a0_ref[...] + a1_ref[...]
    out_ref[...] = (
        jnp.dot(nf_ref[...], cw1_ref[...], preferred_element_type=jnp.float32)
        + jnp.dot(agg, cw2_ref[...], preferred_element_type=jnp.float32)
        + cb_ref[...]
    )


def _out_tc(nf, a0, a1, cw1, cw2, cb2):
    nblk = N_NODES // _OUT_B
    return pl.pallas_call(
        _out_body,
        grid=(nblk,),
        in_specs=[
            pl.BlockSpec((_OUT_B, HIDDEN), lambda i: (i, 0)),
            pl.BlockSpec((_OUT_B, HIDDEN), lambda i: (i, 0)),
            pl.BlockSpec((_OUT_B, HIDDEN), lambda i: (i, 0)),
            pl.BlockSpec((HIDDEN, HIDDEN), lambda i: (0, 0)),
            pl.BlockSpec((HIDDEN, HIDDEN), lambda i: (0, 0)),
            pl.BlockSpec((1, HIDDEN), lambda i: (0, 0)),
        ],
        out_specs=pl.BlockSpec((_OUT_B, HIDDEN), lambda i: (i, 0)),
        out_shape=jax.ShapeDtypeStruct((N_NODES, HIDDEN), jnp.float32),
        compiler_params=pltpu.CompilerParams(
            dimension_semantics=("arbitrary",),
        ),
    )(nf, a0, a1, cw1, cw2, cb2)


def kernel(z, node_feat, edge_index, edge_weight, edge_attr, emb_table,
           dp_w, dp_b, comb_w, comb_b):
    z = z.astype(jnp.int32)
    src = edge_index[0].astype(jnp.int32)
    dst = edge_index[1].astype(jnp.int32)

    zdst = _zdst_sc(z, dst.reshape(NW, EPW)).reshape(N_EDGES)

    ew2 = edge_weight.reshape(N_EDGES, 1)
    zd2 = zdst.reshape(N_EDGES, 1)
    dpw_t = dp_w.T
    dpb2 = dp_b.reshape(1, HIDDEN)
    emb_pad = jnp.zeros((128, HIDDEN), emb_table.dtype).at[:emb_table.shape[0]].set(emb_table)
    msg = _msg_tc(edge_attr, ew2, zd2, dpw_t, dpb2, emb_pad)

    src3 = src.reshape(NW, JB, BL)
    zeros = jnp.zeros((NPAD, HIDDEN), jnp.float32)
    parts = _scatter_sc(msg, src3, zeros)

    cwt = comb_w.T
    out = _out_tc(node_feat, parts[:N_NODES], parts[NPAD:NPAD + N_NODES],
                  cwt[:HIDDEN], cwt[HIDDEN:], comb_b.reshape(1, HIDDEN))
    return out


# speedup vs baseline: 3.8740x; 2.3128x over previous
import functools

import jax
import jax.numpy as jnp
from jax import lax
from jax.experimental import pallas as pl
from jax.experimental.pallas import tpu as pltpu
from jax.experimental.pallas import tpu_sc as plsc

N_NODES = 10000
N_EDGES = 320000
HIDDEN = 128
NUM_RBF = 64
CUTOFF = 5.0

NC = 2
NS = 16
NW = NC * NS
EPW = N_EDGES // NW
BL = 80
JB = EPW // BL
NPAD = 10240
ROWS_PER_TILE = NPAD // NS

@functools.lru_cache(maxsize=None)
def _build_zdst():
    mesh = plsc.VectorSubcoreMesh(core_axis_name="c", subcore_axis_name="s",
                                  num_cores=NC, num_subcores=NS)

    @functools.partial(
        pl.kernel,
        out_type=jax.ShapeDtypeStruct((NW, EPW), jnp.int32),
        mesh=mesh,
        scratch_types=[
            pltpu.VMEM((N_NODES,), jnp.int32),
            pltpu.VMEM((EPW,), jnp.int32),
            pltpu.VMEM((EPW,), jnp.int32),
        ],
        compiler_params=pltpu.CompilerParams(needs_layout_passes=False),
    )
    def zdst_kernel(z_hbm, dst_hbm, out_hbm, z_v, dst_v, out_v):
        wid = lax.axis_index("s") * NC + lax.axis_index("c")
        pltpu.sync_copy(z_hbm, z_v)
        pltpu.sync_copy(dst_hbm.at[wid], dst_v)

        def body(i, carry):
            idx = dst_v[pl.ds(i * 16, 16)]
            out_v[pl.ds(i * 16, 16)] = plsc.load_gather(z_v, [idx])
            return carry

        lax.fori_loop(0, EPW // 16, body, 0)
        pltpu.sync_copy(out_v, out_hbm.at[wid])

    return zdst_kernel


def _zdst_sc(z, dst):
    return _build_zdst()(z, dst)


_MSG_B = 2560


def _msg_body(ea_ref, ew_ref, zd_ref, dpw_ref, dpb_ref, embT_ref, out_ref):
    w = jnp.dot(ea_ref[...], dpw_ref[...], preferred_element_type=jnp.float32)
    ew = ew_ref[0]
    c = jnp.where(ew < CUTOFF, 0.5 * (jnp.cos(ew * (jnp.pi / CUTOFF)) + 1.0), 0.0)
    zd = zd_ref[0]
    ohcT = jnp.where(
        lax.broadcasted_iota(jnp.int32, (128, _MSG_B), 0) == zd, c, 0.0)
    gcT = jnp.dot(embT_ref[...], ohcT, preferred_element_type=jnp.float32)
    out_ref[...] = (w + dpb_ref[...]) * gcT.T


def _msg_tc(ea, ew3, zd3, dpw_t, dpb2, embT):
    nblk = N_EDGES // _MSG_B
    return pl.pallas_call(
        _msg_body,
        grid=(nblk,),
        in_specs=[
            pl.BlockSpec((_MSG_B, NUM_RBF), lambda i: (i, 0)),
            pl.BlockSpec((1, 1, _MSG_B), lambda i: (i, 0, 0)),
            pl.BlockSpec((1, 1, _MSG_B), lambda i: (i, 0, 0)),
            pl.BlockSpec((NUM_RBF, HIDDEN), lambda i: (0, 0)),
            pl.BlockSpec((1, HIDDEN), lambda i: (0, 0)),
            pl.BlockSpec((128, HIDDEN), lambda i: (0, 0)),
        ],
        out_specs=pl.BlockSpec((_MSG_B, HIDDEN), lambda i: (i, 0)),
        out_shape=jax.ShapeDtypeStruct((N_EDGES, HIDDEN), jnp.float32),
        compiler_params=pltpu.CompilerParams(
            dimension_semantics=("arbitrary",),
        ),
    )(ea, ew3, zd3, dpw_t, dpb2, embT)


@functools.lru_cache(maxsize=None)
def _build_scatter():
    mesh = plsc.VectorSubcoreMesh(core_axis_name="c", subcore_axis_name="s",
                                  num_cores=NC, num_subcores=NS)

    @functools.partial(
        pl.kernel,
        out_type=jax.ShapeDtypeStruct((NC * NPAD, HIDDEN), jnp.float32),
        mesh=mesh,
        scratch_types=[
            pltpu.VMEM_SHARED((NPAD, HIDDEN), jnp.float32),
            pltpu.VMEM((JB, BL), jnp.int32),
            pltpu.VMEM((BL, HIDDEN), jnp.float32),
        ],
        compiler_params=pltpu.CompilerParams(needs_layout_passes=False),
    )
    def scatter_kernel(msg_hbm, src_hbm, zeros_hbm, out_hbm, agg_sh, idx_v, buf):
        c = lax.axis_index("c")
        s = lax.axis_index("s")
        wid = s * NC + c
        rb = s * ROWS_PER_TILE
        pltpu.sync_copy(zeros_hbm.at[pl.ds(rb, ROWS_PER_TILE)],
                        agg_sh.at[pl.ds(rb, ROWS_PER_TILE)])
        pltpu.sync_copy(src_hbm.at[wid], idx_v)
        plsc.subcore_barrier()
        ebase = wid * EPW

        def body(j, carry):
            pltpu.sync_copy(msg_hbm.at[pl.ds(ebase + j * BL, BL)], buf)
            pltpu.sync_copy(buf, agg_sh.at[idx_v.at[j]], add=True)
            return carry

        lax.fori_loop(0, JB, body, 0)
        plsc.subcore_barrier()
        pltpu.sync_copy(agg_sh.at[pl.ds(rb, ROWS_PER_TILE)],
                        out_hbm.at[pl.ds(c * NPAD + rb, ROWS_PER_TILE)])

    return scatter_kernel


def _scatter_sc(msg, src3, zeros):
    return _build_scatter()(msg, src3, zeros)


_OUT_B = 1000


def _out_body(nf_ref, a0_ref, a1_ref, cw1_ref, cw2_ref, cb_ref, out_ref):
    agg = a0_ref[...] + a1_ref[...]
    out_ref[...] = (
        jnp.dot(nf_ref[...], cw1_ref[...], preferred_element_type=jnp.float32)
        + jnp.dot(agg, cw2_ref[...], preferred_element_type=jnp.float32)
        + cb_ref[...]
    )


def _out_tc(nf, a0, a1, cw1, cw2, cb2):
    nblk = N_NODES // _OUT_B
    return pl.pallas_call(
        _out_body,
        grid=(nblk,),
        in_specs=[
            pl.BlockSpec((_OUT_B, HIDDEN), lambda i: (i, 0)),
            pl.BlockSpec((_OUT_B, HIDDEN), lambda i: (i, 0)),
            pl.BlockSpec((_OUT_B, HIDDEN), lambda i: (i, 0)),
            pl.BlockSpec((HIDDEN, HIDDEN), lambda i: (0, 0)),
            pl.BlockSpec((HIDDEN, HIDDEN), lambda i: (0, 0)),
            pl.BlockSpec((1, HIDDEN), lambda i: (0, 0)),
        ],
        out_specs=pl.BlockSpec((_OUT_B, HIDDEN), lambda i: (i, 0)),
        out_shape=jax.ShapeDtypeStruct((N_NODES, HIDDEN), jnp.float32),
        compiler_params=pltpu.CompilerParams(
            dimension_semantics=("arbitrary",),
        ),
    )(nf, a0, a1, cw1, cw2, cb2)


def kernel(z, node_feat, edge_index, edge_weight, edge_attr, emb_table,
           dp_w, dp_b, comb_w, comb_b):
    z = z.astype(jnp.int32)
    src = edge_index[0].astype(jnp.int32)
    dst = edge_index[1].astype(jnp.int32)

    zdst = _zdst_sc(z, dst.reshape(NW, EPW)).reshape(N_EDGES)

    nblk = N_EDGES // _MSG_B
    ew3 = edge_weight.reshape(nblk, 1, _MSG_B)
    zd3 = zdst.reshape(nblk, 1, _MSG_B)
    dpw_t = dp_w.T
    dpb2 = dp_b.reshape(1, HIDDEN)
    embT = jnp.zeros((128, HIDDEN), emb_table.dtype).at[:emb_table.shape[0]].set(emb_table).T
    msg = _msg_tc(edge_attr, ew3, zd3, dpw_t, dpb2, embT)

    src3 = src.reshape(NW, JB, BL)
    zeros = jnp.zeros((NPAD, HIDDEN), jnp.float32)
    parts = _scatter_sc(msg, src3, zeros)

    cwt = comb_w.T
    out = _out_tc(node_feat, parts[:N_NODES], parts[NPAD:NPAD + N_NODES],
                  cwt[:HIDDEN], cwt[HIDDEN:], comb_b.reshape(1, HIDDEN))
    return out


# speedup vs baseline: 4.2584x; 1.0992x over previous
import functools

import jax
import jax.numpy as jnp
from jax import lax
from jax.experimental import pallas as pl
from jax.experimental.pallas import tpu as pltpu
from jax.experimental.pallas import tpu_sc as plsc

N_NODES = 10000
N_EDGES = 320000
HIDDEN = 128
NUM_RBF = 64
CUTOFF = 5.0

NC = 2
NS = 16
NW = NC * NS
EPW = N_EDGES // NW
BL = 80
JB = EPW // BL
NPAD = 10240
ROWS_PER_TILE = NPAD // NS

@functools.lru_cache(maxsize=None)
def _build_zdst():
    mesh = plsc.VectorSubcoreMesh(core_axis_name="c", subcore_axis_name="s",
                                  num_cores=NC, num_subcores=NS)

    @functools.partial(
        pl.kernel,
        out_type=jax.ShapeDtypeStruct((NW, EPW), jnp.int32),
        mesh=mesh,
        scratch_types=[
            pltpu.VMEM((N_NODES,), jnp.int32),
            pltpu.VMEM((EPW,), jnp.int32),
            pltpu.VMEM((EPW,), jnp.int32),
        ],
        compiler_params=pltpu.CompilerParams(needs_layout_passes=False),
    )
    def zdst_kernel(z_hbm, dst_hbm, out_hbm, z_v, dst_v, out_v):
        wid = lax.axis_index("s") * NC + lax.axis_index("c")
        pltpu.sync_copy(z_hbm, z_v)
        pltpu.sync_copy(dst_hbm.at[wid], dst_v)

        def body(i, carry):
            idx = dst_v[pl.ds(i * 16, 16)]
            out_v[pl.ds(i * 16, 16)] = plsc.load_gather(z_v, [idx])
            return carry

        lax.fori_loop(0, EPW // 16, body, 0)
        pltpu.sync_copy(out_v, out_hbm.at[wid])

    return zdst_kernel


def _zdst_sc(z, dst):
    return _build_zdst()(z, dst)


_MSG_B = 2560


def _msg_body(ea_ref, ew_ref, zd_ref, dpw_ref, dpb_ref, embT_ref, out_ref):
    w = jnp.dot(ea_ref[...], dpw_ref[...], preferred_element_type=jnp.float32)
    ew = ew_ref[0]
    c = jnp.where(ew < CUTOFF, 0.5 * (jnp.cos(ew * (jnp.pi / CUTOFF)) + 1.0), 0.0)
    zd = zd_ref[0]
    ohcT = jnp.where(
        lax.broadcasted_iota(jnp.int32, (128, _MSG_B), 0) == zd, c, 0.0)
    gcT = jnp.dot(embT_ref[...], ohcT, preferred_element_type=jnp.float32)
    out_ref[...] = (w + dpb_ref[...]) * gcT.T


def _msg_tc(ea, ew3, zd3, dpw_t, dpb2, embT):
    nblk = N_EDGES // _MSG_B
    return pl.pallas_call(
        _msg_body,
        grid=(nblk,),
        in_specs=[
            pl.BlockSpec((_MSG_B, NUM_RBF), lambda i: (i, 0)),
            pl.BlockSpec((1, 1, _MSG_B), lambda i: (i, 0, 0)),
            pl.BlockSpec((1, 1, _MSG_B), lambda i: (i, 0, 0)),
            pl.BlockSpec((NUM_RBF, HIDDEN), lambda i: (0, 0)),
            pl.BlockSpec((1, HIDDEN), lambda i: (0, 0)),
            pl.BlockSpec((128, HIDDEN), lambda i: (0, 0)),
        ],
        out_specs=pl.BlockSpec((_MSG_B, HIDDEN), lambda i: (i, 0)),
        out_shape=jax.ShapeDtypeStruct((N_EDGES, HIDDEN), jnp.float32),
        compiler_params=pltpu.CompilerParams(
            dimension_semantics=("arbitrary",),
        ),
    )(ea, ew3, zd3, dpw_t, dpb2, embT)


@functools.lru_cache(maxsize=None)
def _build_scatter():
    mesh = plsc.VectorSubcoreMesh(core_axis_name="c", subcore_axis_name="s",
                                  num_cores=NC, num_subcores=NS)

    @functools.partial(
        pl.kernel,
        out_type=jax.ShapeDtypeStruct((NC * NPAD, HIDDEN), jnp.float32),
        mesh=mesh,
        scratch_types=[
            pltpu.VMEM_SHARED((NPAD, HIDDEN), jnp.float32),
            pltpu.VMEM((JB, BL), jnp.int32),
            pltpu.VMEM((BL, HIDDEN), jnp.float32),
            pltpu.VMEM((BL, HIDDEN), jnp.float32),
            pltpu.SemaphoreType.DMA,
            pltpu.SemaphoreType.DMA,
            pltpu.SemaphoreType.DMA,
            pltpu.SemaphoreType.DMA,
        ],
    )
    def scatter_kernel(msg_hbm, src_hbm, zeros_hbm, out_hbm, agg_sh,
                       idx_v, buf0, buf1, lsem0, lsem1, ssem0, ssem1):
        c = lax.axis_index("c")
        s = lax.axis_index("s")
        wid = s * NC + c
        rb = s * ROWS_PER_TILE
        ebase = wid * EPW

        def load(j, buf, lsem):
            return pltpu.async_copy(msg_hbm.at[pl.ds(ebase + j * BL, BL)],
                                    buf, lsem)

        load(0, buf0, lsem0)
        load(1, buf1, lsem1)
        pltpu.sync_copy(src_hbm.at[wid], idx_v)
        pltpu.sync_copy(zeros_hbm.at[pl.ds(rb, ROWS_PER_TILE)],
                        agg_sh.at[pl.ds(rb, ROWS_PER_TILE)])
        plsc.subcore_barrier()

        def drain_load(buf, lsem):
            pltpu.make_async_copy(msg_hbm.at[pl.ds(ebase, BL)], buf, lsem).wait()

        def body(g, carry):
            j0 = g * 2
            j1 = j0 + 1
            drain_load(buf0, lsem0)
            sc0 = pltpu.async_copy(buf0, agg_sh.at[idx_v.at[j0]], ssem0,
                                   add=True)
            drain_load(buf1, lsem1)
            sc1 = pltpu.async_copy(buf1, agg_sh.at[idx_v.at[j1]], ssem1,
                                   add=True)
            sc0.wait()
            load(j0 + 2, buf0, lsem0)

            @pl.when(j1 + 2 < JB)
            def _():
                sc1.wait()
                load(j1 + 2, buf1, lsem1)

            return carry

        lax.fori_loop(0, (JB - 1) // 2, body, 0)
        drain_load(buf0, lsem0)
        pltpu.make_async_copy(buf1, agg_sh.at[idx_v.at[JB - 2]], ssem1).wait()
        pltpu.sync_copy(buf0, agg_sh.at[idx_v.at[JB - 1]], add=True)
        plsc.subcore_barrier()
        pltpu.sync_copy(agg_sh.at[pl.ds(rb, ROWS_PER_TILE)],
                        out_hbm.at[pl.ds(c * NPAD + rb, ROWS_PER_TILE)])

    return scatter_kernel


def _scatter_sc(msg, src3, zeros):
    return _build_scatter()(msg, src3, zeros)


_OUT_B = 1000


def _out_body(nf_ref, a0_ref, a1_ref, cw1_ref, cw2_ref, cb_ref, out_ref):
    agg = a0_ref[...] + a1_ref[...]
    out_ref[...] = (
        jnp.dot(nf_ref[...], cw1_ref[...], preferred_element_type=jnp.float32)
        + jnp.dot(agg, cw2_ref[...], preferred_element_type=jnp.float32)
        + cb_ref[...]
    )


def _out_tc(nf, a0, a1, cw1, cw2, cb2):
    nblk = N_NODES // _OUT_B
    return pl.pallas_call(
        _out_body,
        grid=(nblk,),
        in_specs=[
            pl.BlockSpec((_OUT_B, HIDDEN), lambda i: (i, 0)),
            pl.BlockSpec((_OUT_B, HIDDEN), lambda i: (i, 0)),
            pl.BlockSpec((_OUT_B, HIDDEN), lambda i: (i, 0)),
            pl.BlockSpec((HIDDEN, HIDDEN), lambda i: (0, 0)),
            pl.BlockSpec((HIDDEN, HIDDEN), lambda i: (0, 0)),
            pl.BlockSpec((1, HIDDEN), lambda i: (0, 0)),
        ],
        out_specs=pl.BlockSpec((_OUT_B, HIDDEN), lambda i: (i, 0)),
        out_shape=jax.ShapeDtypeStruct((N_NODES, HIDDEN), jnp.float32),
        compiler_params=pltpu.CompilerParams(
            dimension_semantics=("arbitrary",),
        ),
    )(nf, a0, a1, cw1, cw2, cb2)


def kernel(z, node_feat, edge_index, edge_weight, edge_attr, emb_table,
           dp_w, dp_b, comb_w, comb_b):
    z = z.astype(jnp.int32)
    src = edge_index[0].astype(jnp.int32)
    dst = edge_index[1].astype(jnp.int32)

    zdst = _zdst_sc(z, dst.reshape(NW, EPW)).reshape(N_EDGES)

    nblk = N_EDGES // _MSG_B
    ew3 = edge_weight.reshape(nblk, 1, _MSG_B)
    zd3 = zdst.reshape(nblk, 1, _MSG_B)
    dpw_t = dp_w.T
    dpb2 = dp_b.reshape(1, HIDDEN)
    embT = jnp.zeros((128, HIDDEN), emb_table.dtype).at[:emb_table.shape[0]].set(emb_table).T
    msg = _msg_tc(edge_attr, ew3, zd3, dpw_t, dpb2, embT)

    src3 = src.reshape(NW, JB, BL)
    zeros = jnp.zeros((NPAD, HIDDEN), jnp.float32)
    parts = _scatter_sc(msg, src3, zeros)

    cwt = comb_w.T
    out = _out_tc(node_feat, parts[:N_NODES], parts[NPAD:NPAD + N_NODES],
                  cwt[:HIDDEN], cwt[HIDDEN:], comb_b.reshape(1, HIDDEN))
    return out


# speedup vs baseline: 4.6593x; 1.0941x over previous
import functools

import jax
import jax.numpy as jnp
from jax import lax
from jax.experimental import pallas as pl
from jax.experimental.pallas import tpu as pltpu
from jax.experimental.pallas import tpu_sc as plsc

N_NODES = 10000
N_EDGES = 320000
HIDDEN = 128
NUM_RBF = 64
CUTOFF = 5.0

NC = 2
NS = 16
NW = NC * NS
EPW = N_EDGES // NW
BL = 80
JB = EPW // BL
NPAD = 10240
ROWS_PER_TILE = NPAD // NS

@functools.lru_cache(maxsize=None)
def _build_zdst():
    mesh = plsc.VectorSubcoreMesh(core_axis_name="c", subcore_axis_name="s",
                                  num_cores=NC, num_subcores=NS)

    @functools.partial(
        pl.kernel,
        out_type=jax.ShapeDtypeStruct((NW, EPW), jnp.int32),
        mesh=mesh,
        scratch_types=[
            pltpu.VMEM((N_NODES,), jnp.int32),
            pltpu.VMEM((EPW,), jnp.int32),
            pltpu.VMEM((EPW,), jnp.int32),
        ],
        compiler_params=pltpu.CompilerParams(needs_layout_passes=False),
    )
    def zdst_kernel(z_hbm, dst_hbm, out_hbm, z_v, dst_v, out_v):
        wid = lax.axis_index("s") * NC + lax.axis_index("c")
        pltpu.sync_copy(z_hbm, z_v)
        pltpu.sync_copy(dst_hbm.at[wid], dst_v)

        def body(i, carry):
            idx = dst_v[pl.ds(i * 16, 16)]
            out_v[pl.ds(i * 16, 16)] = plsc.load_gather(z_v, [idx])
            return carry

        lax.fori_loop(0, EPW // 16, body, 0)
        pltpu.sync_copy(out_v, out_hbm.at[wid])

    return zdst_kernel


def _zdst_sc(z, dst):
    return _build_zdst()(z, dst)


_MSG_B = 2560


def _msg_body(ea_ref, ew_ref, zd_ref, dpw_ref, dpb_ref, embT_ref, out_ref):
    w = jnp.dot(ea_ref[...], dpw_ref[...], preferred_element_type=jnp.float32)
    ew = ew_ref[0]
    c = jnp.where(ew < CUTOFF, 0.5 * (jnp.cos(ew * (jnp.pi / CUTOFF)) + 1.0), 0.0)
    zd = zd_ref[0]
    ohcT = jnp.where(
        lax.broadcasted_iota(jnp.int32, (128, _MSG_B), 0) == zd, c, 0.0)
    gcT = jnp.dot(embT_ref[...], ohcT, preferred_element_type=jnp.float32)
    out_ref[...] = (w + dpb_ref[...]) * gcT.T


def _msg_tc(ea, ew3, zd3, dpw_t, dpb2, embT, off, nb):
    return pl.pallas_call(
        _msg_body,
        grid=(nb,),
        in_specs=[
            pl.BlockSpec((_MSG_B, NUM_RBF), lambda i: (i + off, 0)),
            pl.BlockSpec((1, 1, _MSG_B), lambda i: (i + off, 0, 0)),
            pl.BlockSpec((1, 1, _MSG_B), lambda i: (i + off, 0, 0)),
            pl.BlockSpec((NUM_RBF, HIDDEN), lambda i: (0, 0)),
            pl.BlockSpec((1, HIDDEN), lambda i: (0, 0)),
            pl.BlockSpec((128, HIDDEN), lambda i: (0, 0)),
        ],
        out_specs=pl.BlockSpec((_MSG_B, HIDDEN), lambda i: (i, 0)),
        out_shape=jax.ShapeDtypeStruct((nb * _MSG_B, HIDDEN), jnp.float32),
        compiler_params=pltpu.CompilerParams(
            dimension_semantics=("arbitrary",),
        ),
    )(ea, ew3, zd3, dpw_t, dpb2, embT)


@functools.lru_cache(maxsize=None)
def _build_scatter(jb):
    epw = jb * BL
    mesh = plsc.VectorSubcoreMesh(core_axis_name="c", subcore_axis_name="s",
                                  num_cores=NC, num_subcores=NS)

    @functools.partial(
        pl.kernel,
        out_type=jax.ShapeDtypeStruct((NC * NPAD, HIDDEN), jnp.float32),
        mesh=mesh,
        scratch_types=[
            pltpu.VMEM_SHARED((NPAD, HIDDEN), jnp.float32),
            pltpu.VMEM((jb, BL), jnp.int32),
            pltpu.VMEM((BL, HIDDEN), jnp.float32),
            pltpu.VMEM((BL, HIDDEN), jnp.float32),
            pltpu.SemaphoreType.DMA,
            pltpu.SemaphoreType.DMA,
            pltpu.SemaphoreType.DMA,
            pltpu.SemaphoreType.DMA,
        ],
    )
    def scatter_kernel(msg_hbm, src_hbm, zeros_hbm, out_hbm, agg_sh,
                       idx_v, buf0, buf1, lsem0, lsem1, ssem0, ssem1):
        c = lax.axis_index("c")
        s = lax.axis_index("s")
        wid = s * NC + c
        rb = s * ROWS_PER_TILE
        ebase = wid * epw

        def load(j, buf, lsem):
            return pltpu.async_copy(msg_hbm.at[pl.ds(ebase + j * BL, BL)],
                                    buf, lsem)

        load(0, buf0, lsem0)
        load(1, buf1, lsem1)
        pltpu.sync_copy(src_hbm.at[wid], idx_v)
        pltpu.sync_copy(zeros_hbm.at[pl.ds(rb, ROWS_PER_TILE)],
                        agg_sh.at[pl.ds(rb, ROWS_PER_TILE)])
        plsc.subcore_barrier()

        def drain_load(buf, lsem):
            pltpu.make_async_copy(msg_hbm.at[pl.ds(ebase, BL)], buf, lsem).wait()

        def body(g, carry):
            j0 = g * 2
            j1 = j0 + 1
            drain_load(buf0, lsem0)
            sc0 = pltpu.async_copy(buf0, agg_sh.at[idx_v.at[j0]], ssem0,
                                   add=True)
            drain_load(buf1, lsem1)
            sc1 = pltpu.async_copy(buf1, agg_sh.at[idx_v.at[j1]], ssem1,
                                   add=True)
            sc0.wait()
            load(j0 + 2, buf0, lsem0)

            @pl.when(j1 + 2 < jb)
            def _():
                sc1.wait()
                load(j1 + 2, buf1, lsem1)

            return carry

        if jb % 2 == 1:
            lax.fori_loop(0, (jb - 1) // 2, body, 0)
            drain_load(buf0, lsem0)
            pltpu.make_async_copy(buf1, agg_sh.at[idx_v.at[jb - 2]], ssem1).wait()
            pltpu.sync_copy(buf0, agg_sh.at[idx_v.at[jb - 1]], add=True)
        else:
            lax.fori_loop(0, jb // 2 - 1, body, 0)
            drain_load(buf0, lsem0)
            pltpu.sync_copy(buf0, agg_sh.at[idx_v.at[jb - 2]], add=True)
            drain_load(buf1, lsem1)
            pltpu.sync_copy(buf1, agg_sh.at[idx_v.at[jb - 1]], add=True)
        plsc.subcore_barrier()
        pltpu.sync_copy(agg_sh.at[pl.ds(rb, ROWS_PER_TILE)],
                        out_hbm.at[pl.ds(c * NPAD + rb, ROWS_PER_TILE)])

    return scatter_kernel


def _scatter_sc(msg, src3, zeros, jb):
    return _build_scatter(jb)(msg, src3, zeros)


_OUT_B = 1000


def _out_body(nf_ref, a0_ref, a1_ref, a2_ref, a3_ref, cw1_ref, cw2_ref,
              cb_ref, out_ref):
    agg = (a0_ref[...] + a1_ref[...]) + (a2_ref[...] + a3_ref[...])
    out_ref[...] = (
        jnp.dot(nf_ref[...], cw1_ref[...], preferred_element_type=jnp.float32)
        + jnp.dot(agg, cw2_ref[...], preferred_element_type=jnp.float32)
        + cb_ref[...]
    )


def _out_tc(nf, a0, a1, a2, a3, cw1, cw2, cb2):
    nblk = N_NODES // _OUT_B
    return pl.pallas_call(
        _out_body,
        grid=(nblk,),
        in_specs=[
            pl.BlockSpec((_OUT_B, HIDDEN), lambda i: (i, 0)),
            pl.BlockSpec((_OUT_B, HIDDEN), lambda i: (i, 0)),
            pl.BlockSpec((_OUT_B, HIDDEN), lambda i: (i, 0)),
            pl.BlockSpec((_OUT_B, HIDDEN), lambda i: (i, 0)),
            pl.BlockSpec((_OUT_B, HIDDEN), lambda i: (i, 0)),
            pl.BlockSpec((HIDDEN, HIDDEN), lambda i: (0, 0)),
            pl.BlockSpec((HIDDEN, HIDDEN), lambda i: (0, 0)),
            pl.BlockSpec((1, HIDDEN), lambda i: (0, 0)),
        ],
        out_specs=pl.BlockSpec((_OUT_B, HIDDEN), lambda i: (i, 0)),
        out_shape=jax.ShapeDtypeStruct((N_NODES, HIDDEN), jnp.float32),
        compiler_params=pltpu.CompilerParams(
            dimension_semantics=("arbitrary",),
        ),
    )(nf, a0, a1, a2, a3, cw1, cw2, cb2)


def kernel(z, node_feat, edge_index, edge_weight, edge_attr, emb_table,
           dp_w, dp_b, comb_w, comb_b):
    z = z.astype(jnp.int32)
    src = edge_index[0].astype(jnp.int32)
    dst = edge_index[1].astype(jnp.int32)

    zdst = _zdst_sc(z, dst.reshape(NW, EPW)).reshape(N_EDGES)

    nblk = N_EDGES // _MSG_B
    ew3 = edge_weight.reshape(nblk, 1, _MSG_B)
    zd3 = zdst.reshape(nblk, 1, _MSG_B)
    dpw_t = dp_w.T
    dpb2 = dp_b.reshape(1, HIDDEN)
    embT = jnp.zeros((128, HIDDEN), emb_table.dtype).at[:emb_table.shape[0]].set(emb_table).T

    nb0 = 63
    nb1 = nblk - nb0
    e0 = nb0 * _MSG_B
    jb0 = e0 // (NW * BL)
    jb1 = (N_EDGES - e0) // (NW * BL)
    zeros = jnp.zeros((NPAD, HIDDEN), jnp.float32)

    msg0 = _msg_tc(edge_attr, ew3, zd3, dpw_t, dpb2, embT, 0, nb0)
    msg1 = _msg_tc(edge_attr, ew3, zd3, dpw_t, dpb2, embT, nb0, nb1)
    parts0 = _scatter_sc(msg0, src[:e0].reshape(NW, jb0, BL), zeros, jb0)
    parts1 = _scatter_sc(msg1, src[e0:].reshape(NW, jb1, BL), zeros, jb1)

    cwt = comb_w.T
    out = _out_tc(node_feat,
                  parts0[:N_NODES], parts0[NPAD:NPAD + N_NODES],
                  parts1[:N_NODES], parts1[NPAD:NPAD + N_NODES],
                  cwt[:HIDDEN], cwt[HIDDEN:], comb_b.reshape(1, HIDDEN))
    return out


# speedup vs baseline: 5.4896x; 1.1782x over previous
import functools

import jax
import jax.numpy as jnp
from jax import lax
from jax.experimental import pallas as pl
from jax.experimental.pallas import tpu as pltpu
from jax.experimental.pallas import tpu_sc as plsc

N_NODES = 10000
N_EDGES = 320000
HIDDEN = 128
NUM_RBF = 64
CUTOFF = 5.0

NC = 2
NS = 16
NW = NC * NS
EPW = N_EDGES // NW
BL = 80
JB = EPW // BL
NPAD = 10240
ROWS_PER_TILE = NPAD // NS

@functools.lru_cache(maxsize=None)
def _build_zdst():
    mesh = plsc.VectorSubcoreMesh(core_axis_name="c", subcore_axis_name="s",
                                  num_cores=NC, num_subcores=NS)

    @functools.partial(
        pl.kernel,
        out_type=jax.ShapeDtypeStruct((NW, EPW), jnp.int32),
        mesh=mesh,
        scratch_types=[
            pltpu.VMEM((N_NODES,), jnp.int32),
            pltpu.VMEM((EPW,), jnp.int32),
            pltpu.VMEM((EPW,), jnp.int32),
        ],
        compiler_params=pltpu.CompilerParams(needs_layout_passes=False),
    )
    def zdst_kernel(z_hbm, dst_hbm, out_hbm, z_v, dst_v, out_v):
        wid = lax.axis_index("s") * NC + lax.axis_index("c")
        pltpu.sync_copy(z_hbm, z_v)
        pltpu.sync_copy(dst_hbm.at[wid], dst_v)

        def body(i, carry):
            idx = dst_v[pl.ds(i * 16, 16)]
            out_v[pl.ds(i * 16, 16)] = plsc.load_gather(z_v, [idx])
            return carry

        lax.fori_loop(0, EPW // 16, body, 0)
        pltpu.sync_copy(out_v, out_hbm.at[wid])

    return zdst_kernel


def _zdst_sc(z, dst):
    return _build_zdst()(z, dst)


_MSG_B = 2560


def _msg_body(eaT_ref, ew_ref, zd_ref, dpw_ref, dpb_ref, embT_ref, out_ref):
    wT = jnp.dot(dpw_ref[...], eaT_ref[...], preferred_element_type=jnp.float32)
    ew = ew_ref[0]
    c = jnp.where(ew < CUTOFF, 0.5 * (jnp.cos(ew * (jnp.pi / CUTOFF)) + 1.0), 0.0)
    zd = zd_ref[0]
    ohcT = jnp.where(
        lax.broadcasted_iota(jnp.int32, (128, _MSG_B), 0) == zd, c, 0.0)
    gcT = jnp.dot(embT_ref[...], ohcT, preferred_element_type=jnp.float32)
    out_ref[...] = ((wT + dpb_ref[...]) * gcT).T


def _msg_tc(eaT, ew3, zd3, dpw, dpb_col, embT, off, nb):
    return pl.pallas_call(
        _msg_body,
        grid=(nb,),
        in_specs=[
            pl.BlockSpec((NUM_RBF, _MSG_B), lambda i: (0, i + off)),
            pl.BlockSpec((1, 1, _MSG_B), lambda i: (i + off, 0, 0)),
            pl.BlockSpec((1, 1, _MSG_B), lambda i: (i + off, 0, 0)),
            pl.BlockSpec((HIDDEN, NUM_RBF), lambda i: (0, 0)),
            pl.BlockSpec((HIDDEN, 1), lambda i: (0, 0)),
            pl.BlockSpec((128, HIDDEN), lambda i: (0, 0)),
        ],
        out_specs=pl.BlockSpec((_MSG_B, HIDDEN), lambda i: (i, 0)),
        out_shape=jax.ShapeDtypeStruct((nb * _MSG_B, HIDDEN), jnp.float32),
        compiler_params=pltpu.CompilerParams(
            dimension_semantics=("arbitrary",),
        ),
    )(eaT, ew3, zd3, dpw, dpb_col, embT)


@functools.lru_cache(maxsize=None)
def _build_scatter(jb):
    epw = jb * BL
    mesh = plsc.VectorSubcoreMesh(core_axis_name="c", subcore_axis_name="s",
                                  num_cores=NC, num_subcores=NS)

    @functools.partial(
        pl.kernel,
        out_type=jax.ShapeDtypeStruct((NC * NPAD, HIDDEN), jnp.float32),
        mesh=mesh,
        scratch_types=[
            pltpu.VMEM_SHARED((NPAD, HIDDEN), jnp.float32),
            pltpu.VMEM((jb, BL), jnp.int32),
            pltpu.VMEM((BL, HIDDEN), jnp.float32),
            pltpu.VMEM((BL, HIDDEN), jnp.float32),
            pltpu.SemaphoreType.DMA,
            pltpu.SemaphoreType.DMA,
            pltpu.SemaphoreType.DMA,
            pltpu.SemaphoreType.DMA,
        ],
    )
    def scatter_kernel(msg_hbm, src_hbm, zeros_hbm, out_hbm, agg_sh,
                       idx_v, buf0, buf1, lsem0, lsem1, ssem0, ssem1):
        c = lax.axis_index("c")
        s = lax.axis_index("s")
        wid = s * NC + c
        rb = s * ROWS_PER_TILE
        ebase = wid * epw

        def load(j, buf, lsem):
            return pltpu.async_copy(msg_hbm.at[pl.ds(ebase + j * BL, BL)],
                                    buf, lsem)

        load(0, buf0, lsem0)
        load(1, buf1, lsem1)
        pltpu.sync_copy(src_hbm.at[wid], idx_v)
        pltpu.sync_copy(zeros_hbm.at[pl.ds(rb, ROWS_PER_TILE)],
                        agg_sh.at[pl.ds(rb, ROWS_PER_TILE)])
        plsc.subcore_barrier()

        def drain_load(buf, lsem):
            pltpu.make_async_copy(msg_hbm.at[pl.ds(ebase, BL)], buf, lsem).wait()

        def body(g, carry):
            j0 = g * 2
            j1 = j0 + 1
            drain_load(buf0, lsem0)
            sc0 = pltpu.async_copy(buf0, agg_sh.at[idx_v.at[j0]], ssem0,
                                   add=True)
            drain_load(buf1, lsem1)
            sc1 = pltpu.async_copy(buf1, agg_sh.at[idx_v.at[j1]], ssem1,
                                   add=True)
            sc0.wait()
            load(j0 + 2, buf0, lsem0)

            @pl.when(j1 + 2 < jb)
            def _():
                sc1.wait()
                load(j1 + 2, buf1, lsem1)

            return carry

        if jb % 2 == 1:
            lax.fori_loop(0, (jb - 1) // 2, body, 0)
            drain_load(buf0, lsem0)
            pltpu.make_async_copy(buf1, agg_sh.at[idx_v.at[jb - 2]], ssem1).wait()
            pltpu.sync_copy(buf0, agg_sh.at[idx_v.at[jb - 1]], add=True)
        else:
            lax.fori_loop(0, jb // 2 - 1, body, 0)
            drain_load(buf0, lsem0)
            pltpu.sync_copy(buf0, agg_sh.at[idx_v.at[jb - 2]], add=True)
            drain_load(buf1, lsem1)
            pltpu.sync_copy(buf1, agg_sh.at[idx_v.at[jb - 1]], add=True)
        plsc.subcore_barrier()
        pltpu.sync_copy(agg_sh.at[pl.ds(rb, ROWS_PER_TILE)],
                        out_hbm.at[pl.ds(c * NPAD + rb, ROWS_PER_TILE)])

    return scatter_kernel


def _scatter_sc(msg, src3, zeros, jb):
    return _build_scatter(jb)(msg, src3, zeros)


_OUT_B = 80


def _out_body(nf_ref, a0_ref, a1_ref, a2_ref, a3_ref, cw1_ref, cw2_ref,
              cb_ref, out_ref):
    agg = (a0_ref[...] + a1_ref[...]) + (a2_ref[...] + a3_ref[...])
    out_ref[...] = (
        jnp.dot(nf_ref[...], cw1_ref[...], preferred_element_type=jnp.float32)
        + jnp.dot(agg, cw2_ref[...], preferred_element_type=jnp.float32)
        + cb_ref[...]
    )


def _out_tc(nf, parts0, parts1, cw1, cw2, cb2):
    nblk = N_NODES // _OUT_B
    halfblk = NPAD // _OUT_B
    return pl.pallas_call(
        _out_body,
        grid=(nblk,),
        in_specs=[
            pl.BlockSpec((_OUT_B, HIDDEN), lambda i: (i, 0)),
            pl.BlockSpec((_OUT_B, HIDDEN), lambda i: (i, 0)),
            pl.BlockSpec((_OUT_B, HIDDEN), lambda i: (i + halfblk, 0)),
            pl.BlockSpec((_OUT_B, HIDDEN), lambda i: (i, 0)),
            pl.BlockSpec((_OUT_B, HIDDEN), lambda i: (i + halfblk, 0)),
            pl.BlockSpec((HIDDEN, HIDDEN), lambda i: (0, 0)),
            pl.BlockSpec((HIDDEN, HIDDEN), lambda i: (0, 0)),
            pl.BlockSpec((1, HIDDEN), lambda i: (0, 0)),
        ],
        out_specs=pl.BlockSpec((_OUT_B, HIDDEN), lambda i: (i, 0)),
        out_shape=jax.ShapeDtypeStruct((N_NODES, HIDDEN), jnp.float32),
        compiler_params=pltpu.CompilerParams(
            dimension_semantics=("arbitrary",),
        ),
    )(nf, parts0, parts0, parts1, parts1, cw1, cw2, cb2)


def kernel(z, node_feat, edge_index, edge_weight, edge_attr, emb_table,
           dp_w, dp_b, comb_w, comb_b):
    z = z.astype(jnp.int32)
    src = edge_index[0].astype(jnp.int32)
    dst = edge_index[1].astype(jnp.int32)

    zdst = _zdst_sc(z, dst.reshape(NW, EPW)).reshape(N_EDGES)

    nblk = N_EDGES // _MSG_B
    ew3 = edge_weight.reshape(nblk, 1, _MSG_B)
    zd3 = zdst.reshape(nblk, 1, _MSG_B)
    embT = jnp.zeros((128, HIDDEN), emb_table.dtype).at[:emb_table.shape[0]].set(emb_table).T

    nb0 = 63
    nb1 = nblk - nb0
    e0 = nb0 * _MSG_B
    jb0 = e0 // (NW * BL)
    jb1 = (N_EDGES - e0) // (NW * BL)
    zeros = jnp.zeros((NPAD, HIDDEN), jnp.float32)

    eaT = edge_attr.T
    dpb_col = dp_b.reshape(HIDDEN, 1)
    msg0 = _msg_tc(eaT, ew3, zd3, dp_w, dpb_col, embT, 0, nb0)
    msg1 = _msg_tc(eaT, ew3, zd3, dp_w, dpb_col, embT, nb0, nb1)
    parts0 = _scatter_sc(msg0, src[:e0].reshape(NW, jb0, BL), zeros, jb0)
    parts1 = _scatter_sc(msg1, src[e0:].reshape(NW, jb1, BL), zeros, jb1)

    cwt = comb_w.T
    out = _out_tc(node_feat, parts0, parts1,
                  cwt[:HIDDEN], cwt[HIDDEN:], comb_b.reshape(1, HIDDEN))
    return out
